# Initial kernel scaffold; baseline (speedup 1.0000x reference)
import functools

import jax
import jax.numpy as jnp
from jax import lax
from jax.experimental import pallas as pl
from jax.experimental.pallas import tpu as pltpu
from jax.experimental.pallas import tpu_sc as plsc

NC = 2
NS = 16
LANES = 16
EROW = 128

N = 10000
E = 320000
N_PAD = 10240
ROWS = 2560
E_PAD = ROWS * EROW
RPT = ROWS // (NC * NS)
NPT = N_PAD // NS
DEGW = 16

_MESH = plsc.VectorSubcoreMesh(core_axis_name="c", subcore_axis_name="s")


def _zero_fill(ref, rows, width, value=0.0):
    vec = jnp.full((LANES,), value, jnp.float32)
    for i in range(rows):
        for j in range(width // LANES):
            ref[i, pl.ds(j * LANES, LANES)] = vec


def _make_deg_kernel():
    @functools.partial(
        pl.kernel,
        out_type=jax.ShapeDtypeStruct((NC, N_PAD, DEGW), jnp.float32),
        mesh=_MESH,
        scratch_types=[
            pltpu.VMEM((RPT, EROW), jnp.int32),
            pltpu.VMEM((EROW, DEGW), jnp.float32),
            pltpu.VMEM((LANES, DEGW), jnp.float32),
            pltpu.VMEM_SHARED((N_PAD, DEGW), jnp.float32),
        ],
    )
    def deg_kernel(dst_hbm, out_hbm, dst_v, ones_v, zrow_v, acc_sh):
        cid = lax.axis_index("c")
        sid = lax.axis_index("s")
        wid = cid * NS + sid

        pltpu.sync_copy(dst_hbm.at[pl.ds(wid * RPT, RPT)], dst_v)
        _zero_fill(ones_v, EROW, DEGW, 1.0)
        _zero_fill(zrow_v, LANES, DEGW, 0.0)

        rowbase = sid * NPT

        @pl.loop(0, NPT, step=LANES)
        def _(k):
            pltpu.sync_copy(zrow_v, acc_sh.at[pl.ds(rowbase + k, LANES)])

        plsc.subcore_barrier()

        @pl.loop(0, RPT)
        def _(r):
            pltpu.sync_copy(ones_v, acc_sh.at[dst_v.at[r]], add=True)

        plsc.subcore_barrier()
        pltpu.sync_copy(
            acc_sh.at[pl.ds(rowbase, NPT)],
            out_hbm.at[cid].at[pl.ds(rowbase, NPT)],
        )

    return deg_kernel


def _make_agg_kernel(feat):
    @functools.partial(
        pl.kernel,
        out_type=jax.ShapeDtypeStruct((NC, N_PAD, feat), jnp.float32),
        mesh=_MESH,
        scratch_types=[
            pltpu.VMEM((RPT, EROW), jnp.int32),
            pltpu.VMEM((RPT, EROW), jnp.int32),
            pltpu.VMEM((EROW, feat), jnp.float32),
            pltpu.VMEM((EROW, feat), jnp.float32),
            pltpu.VMEM((LANES, feat), jnp.float32),
            pltpu.VMEM_SHARED((N_PAD, feat), jnp.float32),
            pltpu.SemaphoreType.DMA,
            pltpu.SemaphoreType.DMA,
        ],
    )
    def agg_kernel(src_hbm, dst_hbm, g_hbm, out_hbm,
                   src_v, dst_v, rows_a, rows_b, zrow_v, acc_sh, sem_a, sem_b):
        cid = lax.axis_index("c")
        sid = lax.axis_index("s")
        wid = cid * NS + sid
        base = wid * RPT

        pltpu.sync_copy(src_hbm.at[pl.ds(base, RPT)], src_v)
        pltpu.sync_copy(dst_hbm.at[pl.ds(base, RPT)], dst_v)
        _zero_fill(zrow_v, LANES, feat, 0.0)

        rowbase = sid * NPT

        @pl.loop(0, NPT, step=LANES)
        def _(k):
            pltpu.sync_copy(zrow_v, acc_sh.at[pl.ds(rowbase + k, LANES)])

        plsc.subcore_barrier()

        @pl.loop(0, RPT, step=2)
        def _(r):
            cp_a = pltpu.async_copy(g_hbm.at[src_v.at[r]], rows_a, sem_a)
            cp_b = pltpu.async_copy(g_hbm.at[src_v.at[r + 1]], rows_b, sem_b)
            cp_a.wait()
            pltpu.sync_copy(rows_a, acc_sh.at[dst_v.at[r]], add=True)
            cp_b.wait()
            pltpu.sync_copy(rows_b, acc_sh.at[dst_v.at[r + 1]], add=True)

        plsc.subcore_barrier()
        pltpu.sync_copy(
            acc_sh.at[pl.ds(rowbase, NPT)],
            out_hbm.at[cid].at[pl.ds(rowbase, NPT)],
        )

    return agg_kernel


BR = 640


def _dinv_block(degp):
    deg = degp[0, :, 0] + degp[1, :, 0] + 1.0
    return lax.rsqrt(deg)


def _mm1_body(x_ref, w_ref, o_ref):
    o_ref[...] = jnp.dot(x_ref[...], w_ref[...],
                         preferred_element_type=jnp.float32,
                         precision=lax.Precision.HIGHEST)


def _scale_body(h_ref, degp_ref, o_ref):
    dinv = _dinv_block(degp_ref[...])
    o_ref[...] = h_ref[...] * dinv[:, None]


def _layer_body(p_ref, g_ref, degp_ref, b_ref, w_ref, o_ref):
    dinv = _dinv_block(degp_ref[...])
    s = (p_ref[0] + p_ref[1] + g_ref[...]) * dinv[:, None] + b_ref[...]
    h = jnp.maximum(s, 0.0)
    o_ref[...] = jnp.dot(h, w_ref[...],
                         preferred_element_type=jnp.float32,
                         precision=lax.Precision.HIGHEST) * dinv[:, None]


def _final_body(p_ref, g_ref, degp_ref, b3_ref, x_ref, wr_ref, br_ref, o_ref):
    dinv = _dinv_block(degp_ref[...])
    conv = (p_ref[0] + p_ref[1] + g_ref[...]) * dinv[:, None] + b3_ref[...]
    res = jnp.dot(x_ref[...], wr_ref[...],
                  preferred_element_type=jnp.float32,
                  precision=lax.Precision.HIGHEST) + br_ref[...]
    o_ref[...] = conv + res


def _row_spec(width):
    return pl.BlockSpec((BR, width), lambda i: (i, 0))


def _parts_spec(width):
    return pl.BlockSpec((NC, BR, width), lambda i: (0, i, 0))


def _full_spec(shape):
    nd = len(shape)
    return pl.BlockSpec(shape, lambda i: (0,) * nd)


def _tc_call(body, in_specs, out_width, grid=N_PAD // BR):
    return pl.pallas_call(
        body,
        grid=(grid,),
        in_specs=in_specs,
        out_specs=_row_spec(out_width),
        out_shape=jax.ShapeDtypeStruct((N_PAD, out_width), jnp.float32),
    )


def kernel(x, edge_index, W1, b1, W2, b2, W3, b3, Wr, br):
    d_in = x.shape[1]
    d_hid = W2.shape[0]
    d_out = W3.shape[1]

    src = edge_index[0].astype(jnp.int32)
    dst = edge_index[1].astype(jnp.int32)
    pad = jnp.full((E_PAD - E,), N_PAD - 1, jnp.int32)
    src2 = jnp.concatenate([src, pad]).reshape(ROWS, EROW)
    dst2 = jnp.concatenate([dst, pad]).reshape(ROWS, EROW)
    x_pad = jnp.zeros((N_PAD, d_in), jnp.float32).at[:N].set(x)

    degp = _make_deg_kernel()(dst2)

    h1 = _tc_call(_mm1_body, [_row_spec(d_in), _full_spec((d_in, d_hid))],
                  d_hid)(x_pad, W1)
    g1 = _tc_call(_scale_body, [_row_spec(d_hid), _parts_spec(DEGW)],
                  d_hid)(h1, degp)
    parts1 = _make_agg_kernel(d_hid)(src2, dst2, g1)

    g2 = _tc_call(
        _layer_body,
        [_parts_spec(d_hid), _row_spec(d_hid), _parts_spec(DEGW),
         _full_spec((1, d_hid)), _full_spec((d_hid, d_hid))],
        d_hid)(parts1, g1, degp, b1.reshape(1, -1), W2)
    parts2 = _make_agg_kernel(d_hid)(src2, dst2, g2)

    g3 = _tc_call(
        _layer_body,
        [_parts_spec(d_hid), _row_spec(d_hid), _parts_spec(DEGW),
         _full_spec((1, d_hid)), _full_spec((d_hid, d_out))],
        d_out)(parts2, g2, degp, b2.reshape(1, -1), W3)
    parts3 = _make_agg_kernel(d_out)(src2, dst2, g3)

    out = _tc_call(
        _final_body,
        [_parts_spec(d_out), _row_spec(d_out), _parts_spec(DEGW),
         _full_spec((1, d_out)), _row_spec(d_in), _full_spec((d_in, d_out)),
         _full_spec((1, d_out))],
        d_out)(parts3, g3, degp, b3.reshape(1, -1), x_pad, Wr,
               br.reshape(1, -1))
    return out[:N]


# speedup vs baseline: 7.2280x; 7.2280x over previous
import functools

import jax
import jax.numpy as jnp
from jax import lax
from jax.experimental import pallas as pl
from jax.experimental.pallas import tpu as pltpu
from jax.experimental.pallas import tpu_sc as plsc

NC = 2
NS = 16
LANES = 16
EROW = 128

N = 10000
E = 320000
N_PAD = 10240
ROWS = 2560
E_PAD = ROWS * EROW
RPT = ROWS // (NC * NS)
NPT = N_PAD // NS
DEGW = 128

_MESH = plsc.VectorSubcoreMesh(core_axis_name="c", subcore_axis_name="s")


def _zero_fill(ref, rows, width, value=0.0):
    vec = jnp.full((LANES,), value, jnp.float32)
    for i in range(rows):
        for j in range(width // LANES):
            ref[i, pl.ds(j * LANES, LANES)] = vec


def _make_deg_kernel():
    @functools.partial(
        pl.kernel,
        out_type=jax.ShapeDtypeStruct((NC, N_PAD, DEGW), jnp.float32),
        mesh=_MESH,
        scratch_types=[
            pltpu.VMEM((RPT, EROW), jnp.int32),
            pltpu.VMEM((EROW, DEGW), jnp.float32),
            pltpu.VMEM_SHARED((N_PAD, DEGW), jnp.float32),
        ],
    )
    def deg_kernel(dst_hbm, out_hbm, dst_v, ones_v, acc_sh):
        cid = lax.axis_index("c")
        sid = lax.axis_index("s")
        wid = cid * NS + sid

        pltpu.sync_copy(dst_hbm.at[pl.ds(wid * RPT, RPT)], dst_v)

        _zero_fill(ones_v, EROW, DEGW, 0.0)
        rowbase = sid * NPT

        @pl.loop(0, NPT, step=EROW)
        def _(k):
            pltpu.sync_copy(ones_v, acc_sh.at[pl.ds(rowbase + k, EROW)])

        _zero_fill(ones_v, EROW, DEGW, 1.0)
        plsc.subcore_barrier()

        @pl.loop(0, RPT)
        def _(r):
            pltpu.sync_copy(ones_v, acc_sh.at[dst_v.at[r]], add=True)

        plsc.subcore_barrier()
        pltpu.sync_copy(
            acc_sh.at[pl.ds(rowbase, NPT)],
            out_hbm.at[cid].at[pl.ds(rowbase, NPT)],
        )

    return deg_kernel


CH = 40


def _make_agg_kernel(feat):
    @functools.partial(
        pl.kernel,
        out_type=jax.ShapeDtypeStruct((NC, N_PAD, feat), jnp.float32),
        mesh=_MESH,
        scratch_types=[
            pltpu.VMEM((CH, EROW), jnp.int32),
            pltpu.VMEM((CH, EROW), jnp.int32),
            pltpu.VMEM((EROW, feat), jnp.float32),
            pltpu.VMEM((EROW, feat), jnp.float32),
            pltpu.VMEM_SHARED((N_PAD, feat), jnp.float32),
            pltpu.SemaphoreType.DMA,
            pltpu.SemaphoreType.DMA,
        ],
    )
    def agg_kernel(src_hbm, dst_hbm, g_hbm, out_hbm,
                   src_v, dst_v, rows_a, rows_b, acc_sh, sem_a, sem_b):
        cid = lax.axis_index("c")
        sid = lax.axis_index("s")
        wid = cid * NS + sid
        base = wid * RPT

        _zero_fill(rows_a, EROW, feat, 0.0)
        rowbase = sid * NPT

        @pl.loop(0, NPT, step=EROW)
        def _(k):
            pltpu.sync_copy(rows_a, acc_sh.at[pl.ds(rowbase + k, EROW)])

        plsc.subcore_barrier()

        @pl.loop(0, RPT, step=CH)
        def _(c):
            pltpu.sync_copy(src_hbm.at[pl.ds(base + c, CH)], src_v)
            pltpu.sync_copy(dst_hbm.at[pl.ds(base + c, CH)], dst_v)

            @pl.loop(0, CH, step=2)
            def _(r):
                cp_a = pltpu.async_copy(g_hbm.at[src_v.at[r]], rows_a, sem_a)
                cp_b = pltpu.async_copy(g_hbm.at[src_v.at[r + 1]], rows_b,
                                        sem_b)
                cp_a.wait()
                pltpu.sync_copy(rows_a, acc_sh.at[dst_v.at[r]], add=True)
                cp_b.wait()
                pltpu.sync_copy(rows_b, acc_sh.at[dst_v.at[r + 1]], add=True)

        plsc.subcore_barrier()
        pltpu.sync_copy(
            acc_sh.at[pl.ds(rowbase, NPT)],
            out_hbm.at[cid].at[pl.ds(rowbase, NPT)],
        )

    return agg_kernel


BR = 640


def _dinv_block(degp):
    deg = degp[0, :, 0] + degp[1, :, 0] + 1.0
    return lax.rsqrt(deg)


def _mm1_body(x_ref, w_ref, o_ref):
    o_ref[...] = jnp.dot(x_ref[...], w_ref[...],
                         preferred_element_type=jnp.float32,
                         precision=lax.Precision.HIGHEST)


def _scale_body(h_ref, degp_ref, o_ref):
    dinv = _dinv_block(degp_ref[...])
    o_ref[...] = h_ref[...] * dinv[:, None]


def _layer_body(p_ref, g_ref, degp_ref, b_ref, w_ref, o_ref):
    dinv = _dinv_block(degp_ref[...])
    s = (p_ref[0] + p_ref[1] + g_ref[...]) * dinv[:, None] + b_ref[...]
    h = jnp.maximum(s, 0.0)
    o_ref[...] = jnp.dot(h, w_ref[...],
                         preferred_element_type=jnp.float32,
                         precision=lax.Precision.HIGHEST) * dinv[:, None]


def _layer_scale_body(p_ref, g_ref, degp_ref, b_ref, o_ref):
    dinv = _dinv_block(degp_ref[...])
    s = (p_ref[0] + p_ref[1] + g_ref[...]) * dinv[:, None] + b_ref[...]
    o_ref[...] = jnp.maximum(s, 0.0) * dinv[:, None]


def _final_body(p_ref, g_ref, degp_ref, w3_ref, b3_ref, x_ref, wr_ref,
                br_ref, o_ref):
    dinv = _dinv_block(degp_ref[...])
    t = (p_ref[0] + p_ref[1] + g_ref[...]) * dinv[:, None]
    conv = jnp.dot(t, w3_ref[...],
                   preferred_element_type=jnp.float32,
                   precision=lax.Precision.HIGHEST) + b3_ref[...]
    res = jnp.dot(x_ref[...], wr_ref[...],
                  preferred_element_type=jnp.float32,
                  precision=lax.Precision.HIGHEST) + br_ref[...]
    o_ref[...] = conv + res


def _row_spec(width):
    return pl.BlockSpec((BR, width), lambda i: (i, 0))


def _parts_spec(width):
    return pl.BlockSpec((NC, BR, width), lambda i: (0, i, 0))


def _full_spec(shape):
    nd = len(shape)
    return pl.BlockSpec(shape, lambda i: (0,) * nd)


def _tc_call(body, in_specs, out_width, grid=N_PAD // BR):
    return pl.pallas_call(
        body,
        grid=(grid,),
        in_specs=in_specs,
        out_specs=_row_spec(out_width),
        out_shape=jax.ShapeDtypeStruct((N_PAD, out_width), jnp.float32),
    )


def kernel(x, edge_index, W1, b1, W2, b2, W3, b3, Wr, br):
    d_in = x.shape[1]
    d_hid = W2.shape[0]
    d_out = W3.shape[1]

    src = edge_index[0].astype(jnp.int32)
    dst = edge_index[1].astype(jnp.int32)
    pad = jnp.full((E_PAD - E,), N_PAD - 1, jnp.int32)
    src2 = jnp.concatenate([src, pad]).reshape(ROWS, EROW)
    dst2 = jnp.concatenate([dst, pad]).reshape(ROWS, EROW)
    x_pad = jnp.zeros((N_PAD, d_in), jnp.float32).at[:N].set(x)

    degp = _make_deg_kernel()(dst2)

    h1 = _tc_call(_mm1_body, [_row_spec(d_in), _full_spec((d_in, d_hid))],
                  d_hid)(x_pad, W1)
    g1 = _tc_call(_scale_body, [_row_spec(d_hid), _parts_spec(DEGW)],
                  d_hid)(h1, degp)
    parts1 = _make_agg_kernel(d_hid)(src2, dst2, g1)

    g2 = _tc_call(
        _layer_body,
        [_parts_spec(d_hid), _row_spec(d_hid), _parts_spec(DEGW),
         _full_spec((1, d_hid)), _full_spec((d_hid, d_hid))],
        d_hid)(parts1, g1, degp, b1.reshape(1, -1), W2)
    parts2 = _make_agg_kernel(d_hid)(src2, dst2, g2)

    g2b = _tc_call(
        _layer_scale_body,
        [_parts_spec(d_hid), _row_spec(d_hid), _parts_spec(DEGW),
         _full_spec((1, d_hid))],
        d_hid)(parts2, g2, degp, b2.reshape(1, -1))
    parts3 = _make_agg_kernel(d_hid)(src2, dst2, g2b)

    out = _tc_call(
        _final_body,
        [_parts_spec(d_hid), _row_spec(d_hid), _parts_spec(DEGW),
         _full_spec((d_hid, d_out)), _full_spec((1, d_out)), _row_spec(d_in),
         _full_spec((d_in, d_out)), _full_spec((1, d_out))],
        d_out)(parts3, g2b, degp, W3, b3.reshape(1, -1), x_pad, Wr,
               br.reshape(1, -1))
    return out[:N]


# speedup vs baseline: 19.7124x; 2.7272x over previous
import functools

import jax
import jax.numpy as jnp
from jax import lax
from jax.experimental import pallas as pl
from jax.experimental.pallas import tpu as pltpu
from jax.experimental.pallas import tpu_sc as plsc

NC = 2
NS = 16
LANES = 16
EROW = 128

N = 10000
E = 320000
N_PAD = 10240
ROWS = 2560
E_PAD = ROWS * EROW
RPT = ROWS // (NC * NS)
NPT = N_PAD // NS
DEGW = 128

_MESH = plsc.VectorSubcoreMesh(core_axis_name="c", subcore_axis_name="s")


def _zero_fill(ref, rows, width, value=0.0):
    vec = jnp.full((LANES,), value, jnp.float32)
    for i in range(rows):
        for j in range(width // LANES):
            ref[i, pl.ds(j * LANES, LANES)] = vec


def _make_deg_kernel():
    @functools.partial(
        pl.kernel,
        out_type=jax.ShapeDtypeStruct((NC, N_PAD, DEGW), jnp.float32),
        mesh=_MESH,
        scratch_types=[
            pltpu.VMEM((RPT, EROW), jnp.int32),
            pltpu.VMEM((EROW, DEGW), jnp.float32),
            pltpu.VMEM_SHARED((N_PAD, DEGW), jnp.float32),
        ],
    )
    def deg_kernel(dst_hbm, out_hbm, dst_v, ones_v, acc_sh):
        cid = lax.axis_index("c")
        sid = lax.axis_index("s")
        wid = cid * NS + sid

        pltpu.sync_copy(dst_hbm.at[pl.ds(wid * RPT, RPT)], dst_v)

        _zero_fill(ones_v, EROW, DEGW, 0.0)
        rowbase = sid * NPT

        @pl.loop(0, NPT, step=EROW)
        def _(k):
            pltpu.sync_copy(ones_v, acc_sh.at[pl.ds(rowbase + k, EROW)])

        _zero_fill(ones_v, EROW, DEGW, 1.0)
        plsc.subcore_barrier()

        @pl.loop(0, RPT)
        def _(r):
            pltpu.sync_copy(ones_v, acc_sh.at[dst_v.at[r]], add=True)

        plsc.subcore_barrier()
        pltpu.sync_copy(
            acc_sh.at[pl.ds(rowbase, NPT)],
            out_hbm.at[cid].at[pl.ds(rowbase, NPT)],
        )

    return deg_kernel


CH = 40


def _make_agg_kernel(feat):
    @functools.partial(
        pl.kernel,
        out_type=jax.ShapeDtypeStruct((NC, N_PAD, feat), jnp.float32),
        mesh=_MESH,
        scratch_types=[
            pltpu.VMEM((CH, EROW), jnp.int32),
            pltpu.VMEM((CH, EROW), jnp.int32),
            pltpu.VMEM((EROW, feat), jnp.float32),
            pltpu.VMEM((EROW, feat), jnp.float32),
            pltpu.VMEM_SHARED((N_PAD, feat), jnp.float32),
            pltpu.SemaphoreType.DMA,
            pltpu.SemaphoreType.DMA,
        ],
    )
    def agg_kernel(src_hbm, dst_hbm, g_hbm, out_hbm,
                   src_v, dst_v, rows_a, rows_b, acc_sh, sem_a, sem_b):
        cid = lax.axis_index("c")
        sid = lax.axis_index("s")
        wid = cid * NS + sid
        base = wid * RPT

        _zero_fill(rows_a, EROW, feat, 0.0)
        rowbase = sid * NPT

        @pl.loop(0, NPT, step=EROW)
        def _(k):
            pltpu.sync_copy(rows_a, acc_sh.at[pl.ds(rowbase + k, EROW)])

        plsc.subcore_barrier()

        @pl.loop(0, RPT, step=CH)
        def _(c):
            pltpu.sync_copy(src_hbm.at[pl.ds(base + c, CH)], src_v)
            pltpu.sync_copy(dst_hbm.at[pl.ds(base + c, CH)], dst_v)

            @pl.loop(0, CH, step=2)
            def _(r):
                cp_a = pltpu.async_copy(g_hbm.at[src_v.at[r]], rows_a, sem_a)
                cp_b = pltpu.async_copy(g_hbm.at[src_v.at[r + 1]], rows_b,
                                        sem_b)
                cp_a.wait()
                pltpu.sync_copy(rows_a, acc_sh.at[dst_v.at[r]], add=True)
                cp_b.wait()
                pltpu.sync_copy(rows_b, acc_sh.at[dst_v.at[r + 1]], add=True)

        plsc.subcore_barrier()
        pltpu.sync_copy(
            acc_sh.at[pl.ds(rowbase, NPT)],
            out_hbm.at[cid].at[pl.ds(rowbase, NPT)],
        )

    return agg_kernel


BR = 640


def _dinv_block(degp):
    deg = degp[0, :, 0] + degp[1, :, 0] + 1.0
    return lax.rsqrt(deg)


def _mm1_body(x_ref, w_ref, o_ref):
    o_ref[...] = jnp.dot(x_ref[...], w_ref[...],
                         preferred_element_type=jnp.float32,
                         precision=lax.Precision.HIGHEST)


def _scale_body(h_ref, degp_ref, o_ref):
    dinv = _dinv_block(degp_ref[...])
    o_ref[...] = h_ref[...] * dinv[:, None]


def _layer_body(p_ref, g_ref, degp_ref, b_ref, w_ref, o_ref):
    dinv = _dinv_block(degp_ref[...])
    s = (p_ref[0] + p_ref[1] + g_ref[...]) * dinv[:, None] + b_ref[...]
    h = jnp.maximum(s, 0.0)
    o_ref[...] = jnp.dot(h, w_ref[...],
                         preferred_element_type=jnp.float32,
                         precision=lax.Precision.HIGHEST) * dinv[:, None]


def _layer_scale_body(p_ref, g_ref, degp_ref, b_ref, o_ref):
    dinv = _dinv_block(degp_ref[...])
    s = (p_ref[0] + p_ref[1] + g_ref[...]) * dinv[:, None] + b_ref[...]
    o_ref[...] = jnp.maximum(s, 0.0) * dinv[:, None]


def _final_body(p_ref, g_ref, degp_ref, w3_ref, b3_ref, x_ref, wr_ref,
                br_ref, o_ref):
    dinv = _dinv_block(degp_ref[...])
    t = (p_ref[0] + p_ref[1] + g_ref[...]) * dinv[:, None]
    conv = jnp.dot(t, w3_ref[...],
                   preferred_element_type=jnp.float32,
                   precision=lax.Precision.HIGHEST) + b3_ref[...]
    res = jnp.dot(x_ref[...], wr_ref[...],
                  preferred_element_type=jnp.float32,
                  precision=lax.Precision.HIGHEST) + br_ref[...]
    o_ref[...] = conv + res


def _row_spec(width):
    return pl.BlockSpec((BR, width), lambda i: (i, 0))


def _parts_spec(width):
    return pl.BlockSpec((NC, BR, width), lambda i: (0, i, 0))


def _full_spec(shape):
    nd = len(shape)
    return pl.BlockSpec(shape, lambda i: (0,) * nd)


def _tc_call(body, in_specs, out_width, grid=N_PAD // BR):
    return pl.pallas_call(
        body,
        grid=(grid,),
        in_specs=in_specs,
        out_specs=_row_spec(out_width),
        out_shape=jax.ShapeDtypeStruct((N_PAD, out_width), jnp.float32),
    )


def kernel(x, edge_index, W1, b1, W2, b2, W3, b3, Wr, br):
    d_in = x.shape[1]
    d_hid = W2.shape[0]
    d_out = W3.shape[1]

    src = edge_index[0].astype(jnp.int32)
    dst = edge_index[1].astype(jnp.int32)
    pad = N + (jnp.arange(E_PAD - E, dtype=jnp.int32) % (N_PAD - N))
    src2 = jnp.concatenate([src, pad]).reshape(ROWS, EROW)
    dst2 = jnp.concatenate([dst, pad]).reshape(ROWS, EROW)
    x_pad = jnp.zeros((N_PAD, d_in), jnp.float32).at[:N].set(x)

    degp = _make_deg_kernel()(dst2)

    h1 = _tc_call(_mm1_body, [_row_spec(d_in), _full_spec((d_in, d_hid))],
                  d_hid)(x_pad, W1)
    g1 = _tc_call(_scale_body, [_row_spec(d_hid), _parts_spec(DEGW)],
                  d_hid)(h1, degp)
    parts1 = _make_agg_kernel(d_hid)(src2, dst2, g1)

    g2 = _tc_call(
        _layer_body,
        [_parts_spec(d_hid), _row_spec(d_hid), _parts_spec(DEGW),
         _full_spec((1, d_hid)), _full_spec((d_hid, d_hid))],
        d_hid)(parts1, g1, degp, b1.reshape(1, -1), W2)
    parts2 = _make_agg_kernel(d_hid)(src2, dst2, g2)

    g2b = _tc_call(
        _layer_scale_body,
        [_parts_spec(d_hid), _row_spec(d_hid), _parts_spec(DEGW),
         _full_spec((1, d_hid))],
        d_hid)(parts2, g2, degp, b2.reshape(1, -1))
    parts3 = _make_agg_kernel(d_hid)(src2, dst2, g2b)

    out = _tc_call(
        _final_body,
        [_parts_spec(d_hid), _row_spec(d_hid), _parts_spec(DEGW),
         _full_spec((d_hid, d_out)), _full_spec((1, d_out)), _row_spec(d_in),
         _full_spec((d_in, d_out)), _full_spec((1, d_out))],
        d_out)(parts3, g2b, degp, W3, b3.reshape(1, -1), x_pad, Wr,
               br.reshape(1, -1))
    return out[:N]


# speedup vs baseline: 20.1644x; 1.0229x over previous
import functools

import jax
import jax.numpy as jnp
from jax import lax
from jax.experimental import pallas as pl
from jax.experimental.pallas import tpu as pltpu
from jax.experimental.pallas import tpu_sc as plsc

NC = 2
NS = 16
LANES = 16
EROW = 128

N = 10000
E = 320000
N_PAD = 10240
ROWS = 2560
E_PAD = ROWS * EROW
RPT = ROWS // (NC * NS)
NPT = N_PAD // NS
DEGW = 128

_MESH = plsc.VectorSubcoreMesh(core_axis_name="c", subcore_axis_name="s")


def _zero_fill(ref, rows, width, value=0.0):
    vec = jnp.full((LANES,), value, jnp.float32)
    for i in range(rows):
        for j in range(width // LANES):
            ref[i, pl.ds(j * LANES, LANES)] = vec


def _make_deg_kernel():
    @functools.partial(
        pl.kernel,
        out_type=jax.ShapeDtypeStruct((NC, N_PAD, DEGW), jnp.float32),
        mesh=_MESH,
        scratch_types=[
            pltpu.VMEM((RPT, EROW), jnp.int32),
            pltpu.VMEM((EROW, DEGW), jnp.float32),
            pltpu.VMEM_SHARED((N_PAD, DEGW), jnp.float32),
        ],
    )
    def deg_kernel(dst_hbm, out_hbm, dst_v, ones_v, acc_sh):
        cid = lax.axis_index("c")
        sid = lax.axis_index("s")
        wid = cid * NS + sid

        pltpu.sync_copy(dst_hbm.at[pl.ds(wid * RPT, RPT)], dst_v)

        _zero_fill(ones_v, EROW, DEGW, 0.0)
        rowbase = sid * NPT

        @pl.loop(0, NPT, step=EROW)
        def _(k):
            pltpu.sync_copy(ones_v, acc_sh.at[pl.ds(rowbase + k, EROW)])

        _zero_fill(ones_v, EROW, DEGW, 1.0)
        plsc.subcore_barrier()

        @pl.loop(0, RPT)
        def _(r):
            pltpu.sync_copy(ones_v, acc_sh.at[dst_v.at[r]], add=True)

        plsc.subcore_barrier()
        pltpu.sync_copy(
            acc_sh.at[pl.ds(rowbase, NPT)],
            out_hbm.at[cid].at[pl.ds(rowbase, NPT)],
        )

    return deg_kernel


CH = 40


def _make_agg_kernel(feat):
    @functools.partial(
        pl.kernel,
        out_type=jax.ShapeDtypeStruct((NC, N_PAD, feat), jnp.float32),
        mesh=_MESH,
        scratch_types=[
            pltpu.VMEM((CH, EROW), jnp.int32),
            pltpu.VMEM((CH, EROW), jnp.int32),
            pltpu.VMEM((EROW, feat), jnp.float32),
            pltpu.VMEM((EROW, feat), jnp.float32),
            pltpu.VMEM_SHARED((N_PAD, feat), jnp.float32),
            pltpu.SemaphoreType.DMA,
            pltpu.SemaphoreType.DMA,
            pltpu.SemaphoreType.DMA,
            pltpu.SemaphoreType.DMA,
        ],
    )
    def agg_kernel(src_hbm, dst_hbm, g_hbm, out_hbm,
                   src_v, dst_v, rows_a, rows_b, acc_sh,
                   sem_a, sem_b, sem_sa, sem_sb):
        cid = lax.axis_index("c")
        sid = lax.axis_index("s")
        wid = cid * NS + sid
        base = wid * RPT

        _zero_fill(rows_a, EROW, feat, 0.0)
        rowbase = sid * NPT

        @pl.loop(0, NPT, step=EROW)
        def _(k):
            pltpu.sync_copy(rows_a, acc_sh.at[pl.ds(rowbase + k, EROW)])

        plsc.subcore_barrier()

        def start_gath(r, buf, sem):
            pltpu.make_async_copy(g_hbm.at[src_v.at[r]], buf, sem).start()

        def wait_gath(r, buf, sem):
            pltpu.make_async_copy(g_hbm.at[src_v.at[r]], buf, sem).wait()

        @pl.loop(0, RPT, step=CH)
        def _(c):
            pltpu.sync_copy(src_hbm.at[pl.ds(base + c, CH)], src_v)
            pltpu.sync_copy(dst_hbm.at[pl.ds(base + c, CH)], dst_v)

            start_gath(0, rows_a, sem_a)
            start_gath(1, rows_b, sem_b)

            @pl.loop(0, CH - 2, step=2)
            def _(r):
                wait_gath(r, rows_a, sem_a)
                sc_a = pltpu.async_copy(rows_a, acc_sh.at[dst_v.at[r]],
                                        sem_sa, add=True)
                wait_gath(r + 1, rows_b, sem_b)
                sc_b = pltpu.async_copy(rows_b, acc_sh.at[dst_v.at[r + 1]],
                                        sem_sb, add=True)
                sc_a.wait()
                start_gath(r + 2, rows_a, sem_a)
                sc_b.wait()
                start_gath(r + 3, rows_b, sem_b)

            wait_gath(CH - 2, rows_a, sem_a)
            pltpu.sync_copy(rows_a, acc_sh.at[dst_v.at[CH - 2]], add=True)
            wait_gath(CH - 1, rows_b, sem_b)
            pltpu.sync_copy(rows_b, acc_sh.at[dst_v.at[CH - 1]], add=True)

        plsc.subcore_barrier()
        pltpu.sync_copy(
            acc_sh.at[pl.ds(rowbase, NPT)],
            out_hbm.at[cid].at[pl.ds(rowbase, NPT)],
        )

    return agg_kernel


BR = 640


def _dinv_block(degp):
    deg = degp[0, :, 0] + degp[1, :, 0] + 1.0
    return lax.rsqrt(deg)


def _mm1_body(x_ref, w_ref, o_ref):
    o_ref[...] = jnp.dot(x_ref[...], w_ref[...],
                         preferred_element_type=jnp.float32,
                         precision=lax.Precision.HIGHEST)


def _scale_body(h_ref, degp_ref, o_ref):
    dinv = _dinv_block(degp_ref[...])
    o_ref[...] = h_ref[...] * dinv[:, None]


def _layer_body(p_ref, g_ref, degp_ref, b_ref, w_ref, o_ref):
    dinv = _dinv_block(degp_ref[...])
    s = (p_ref[0] + p_ref[1] + g_ref[...]) * dinv[:, None] + b_ref[...]
    h = jnp.maximum(s, 0.0)
    o_ref[...] = jnp.dot(h, w_ref[...],
                         preferred_element_type=jnp.float32,
                         precision=lax.Precision.HIGHEST) * dinv[:, None]


def _layer_scale_body(p_ref, g_ref, degp_ref, b_ref, o_ref):
    dinv = _dinv_block(degp_ref[...])
    s = (p_ref[0] + p_ref[1] + g_ref[...]) * dinv[:, None] + b_ref[...]
    o_ref[...] = jnp.maximum(s, 0.0) * dinv[:, None]


def _final_body(p_ref, g_ref, degp_ref, w3_ref, b3_ref, x_ref, wr_ref,
                br_ref, o_ref):
    dinv = _dinv_block(degp_ref[...])
    t = (p_ref[0] + p_ref[1] + g_ref[...]) * dinv[:, None]
    conv = jnp.dot(t, w3_ref[...],
                   preferred_element_type=jnp.float32,
                   precision=lax.Precision.HIGHEST) + b3_ref[...]
    res = jnp.dot(x_ref[...], wr_ref[...],
                  preferred_element_type=jnp.float32,
                  precision=lax.Precision.HIGHEST) + br_ref[...]
    o_ref[...] = conv + res


def _row_spec(width):
    return pl.BlockSpec((BR, width), lambda i: (i, 0))


def _parts_spec(width):
    return pl.BlockSpec((NC, BR, width), lambda i: (0, i, 0))


def _full_spec(shape):
    nd = len(shape)
    return pl.BlockSpec(shape, lambda i: (0,) * nd)


def _tc_call(body, in_specs, out_width, grid=N_PAD // BR):
    return pl.pallas_call(
        body,
        grid=(grid,),
        in_specs=in_specs,
        out_specs=_row_spec(out_width),
        out_shape=jax.ShapeDtypeStruct((N_PAD, out_width), jnp.float32),
    )


def kernel(x, edge_index, W1, b1, W2, b2, W3, b3, Wr, br):
    d_in = x.shape[1]
    d_hid = W2.shape[0]
    d_out = W3.shape[1]

    src = edge_index[0].astype(jnp.int32)
    dst = edge_index[1].astype(jnp.int32)
    pad = N + (jnp.arange(E_PAD - E, dtype=jnp.int32) % (N_PAD - N))
    src2 = jnp.concatenate([src, pad]).reshape(ROWS, EROW)
    dst2 = jnp.concatenate([dst, pad]).reshape(ROWS, EROW)
    x_pad = jnp.zeros((N_PAD, d_in), jnp.float32).at[:N].set(x)

    degp = _make_deg_kernel()(dst2)

    h1 = _tc_call(_mm1_body, [_row_spec(d_in), _full_spec((d_in, d_hid))],
                  d_hid)(x_pad, W1)
    g1 = _tc_call(_scale_body, [_row_spec(d_hid), _parts_spec(DEGW)],
                  d_hid)(h1, degp)
    parts1 = _make_agg_kernel(d_hid)(src2, dst2, g1)

    g2 = _tc_call(
        _layer_body,
        [_parts_spec(d_hid), _row_spec(d_hid), _parts_spec(DEGW),
         _full_spec((1, d_hid)), _full_spec((d_hid, d_hid))],
        d_hid)(parts1, g1, degp, b1.reshape(1, -1), W2)
    parts2 = _make_agg_kernel(d_hid)(src2, dst2, g2)

    g2b = _tc_call(
        _layer_scale_body,
        [_parts_spec(d_hid), _row_spec(d_hid), _parts_spec(DEGW),
         _full_spec((1, d_hid))],
        d_hid)(parts2, g2, degp, b2.reshape(1, -1))
    parts3 = _make_agg_kernel(d_hid)(src2, dst2, g2b)

    out = _tc_call(
        _final_body,
        [_parts_spec(d_hid), _row_spec(d_hid), _parts_spec(DEGW),
         _full_spec((d_hid, d_out)), _full_spec((1, d_out)), _row_spec(d_in),
         _full_spec((d_in, d_out)), _full_spec((1, d_out))],
        d_out)(parts3, g2b, degp, W3, b3.reshape(1, -1), x_pad, Wr,
               br.reshape(1, -1))
    return out[:N]


# speedup vs baseline: 22.5122x; 1.1164x over previous
import dataclasses
import functools

import jax
import jax.numpy as jnp
from jax import lax
from jax.experimental import pallas as pl
from jax.experimental.pallas import tpu as pltpu
from jax.experimental.pallas import tpu_sc as plsc

_SC_CP = pltpu.CompilerParams()
if "needs_layout_passes" in pltpu.CompilerParams.__dataclass_fields__:
    _SC_CP = dataclasses.replace(_SC_CP, needs_layout_passes=False)

NC = 2
NS = 16
LANES = 16
EROW = 128

N = 10000
E = 320000
N_PAD = 10240
ROWS = 2560
E_PAD = ROWS * EROW
RPT = ROWS // (NC * NS)
NPT = N_PAD // NS
def _degp_spec():
    return pl.BlockSpec((NC, NS, NPT), lambda i: (0, 0, 0))

_MESH = plsc.VectorSubcoreMesh(core_axis_name="c", subcore_axis_name="s")


def _zero_fill(ref, rows, width, value=0.0):
    vec = jnp.full((LANES,), value, jnp.float32)
    for i in range(rows):
        for j in range(width // LANES):
            ref[i, pl.ds(j * LANES, LANES)] = vec


def _make_deg_kernel():
    @functools.partial(
        pl.kernel,
        out_type=jax.ShapeDtypeStruct((NC, NS, NPT), jnp.float32),
        mesh=_MESH,
        scratch_types=[
            pltpu.VMEM((RPT, EROW), jnp.int32),
            pltpu.VMEM((N_PAD,), jnp.float32),
            pltpu.VMEM((NS, NPT), jnp.float32),
            pltpu.VMEM((NPT,), jnp.float32),
            pltpu.VMEM_SHARED((NS, N_PAD), jnp.float32),
        ],
        compiler_params=_SC_CP,
    )
    def deg_kernel(dst_hbm, out_hbm, dst_v, hist_v, red_v, sum_v, slab_sh):
        cid = lax.axis_index("c")
        sid = lax.axis_index("s")
        wid = cid * NS + sid

        pltpu.sync_copy(dst_hbm.at[pl.ds(wid * RPT, RPT)], dst_v)

        zvec = jnp.zeros((LANES,), jnp.float32)

        @pl.loop(0, N_PAD, step=LANES)
        def _(k):
            hist_v[pl.ds(k, LANES)] = zvec

        ones = jnp.ones((LANES,), jnp.float32)

        @pl.loop(0, RPT)
        def _(r):
            for g in range(EROW // LANES):
                iv = dst_v[r, pl.ds(g * LANES, LANES)]
                plsc.addupdate_scatter(hist_v, [iv], ones)

        pltpu.sync_copy(hist_v, slab_sh.at[sid])
        plsc.subcore_barrier()

        colbase = sid * NPT
        for rr in range(NS):
            pltpu.sync_copy(slab_sh.at[rr].at[pl.ds(colbase, NPT)],
                            red_v.at[rr])
        for c in range(NPT // LANES):
            sl = pl.ds(c * LANES, LANES)
            acc = red_v[0, sl]
            for rr in range(1, NS):
                acc = acc + red_v[rr, sl]
            sum_v[sl] = acc
        pltpu.sync_copy(sum_v, out_hbm.at[cid].at[sid])

    return deg_kernel


CH = 40


def _make_agg_kernel(feat):
    @functools.partial(
        pl.kernel,
        out_type=jax.ShapeDtypeStruct((NC, N_PAD, feat), jnp.float32),
        mesh=_MESH,
        scratch_types=[
            pltpu.VMEM((CH, EROW), jnp.int32),
            pltpu.VMEM((CH, EROW), jnp.int32),
            pltpu.VMEM((EROW, feat), jnp.float32),
            pltpu.VMEM((EROW, feat), jnp.float32),
            pltpu.VMEM_SHARED((N_PAD, feat), jnp.float32),
            pltpu.SemaphoreType.DMA,
            pltpu.SemaphoreType.DMA,
            pltpu.SemaphoreType.DMA,
            pltpu.SemaphoreType.DMA,
        ],
    )
    def agg_kernel(src_hbm, dst_hbm, g_hbm, out_hbm,
                   src_v, dst_v, rows_a, rows_b, acc_sh,
                   sem_a, sem_b, sem_sa, sem_sb):
        cid = lax.axis_index("c")
        sid = lax.axis_index("s")
        wid = cid * NS + sid
        base = wid * RPT

        _zero_fill(rows_a, EROW, feat, 0.0)
        rowbase = sid * NPT

        @pl.loop(0, NPT, step=EROW)
        def _(k):
            pltpu.sync_copy(rows_a, acc_sh.at[pl.ds(rowbase + k, EROW)])

        plsc.subcore_barrier()

        def start_gath(r, buf, sem):
            pltpu.make_async_copy(g_hbm.at[src_v.at[r]], buf, sem).start()

        def wait_gath(r, buf, sem):
            pltpu.make_async_copy(g_hbm.at[src_v.at[r]], buf, sem).wait()

        @pl.loop(0, RPT, step=CH)
        def _(c):
            pltpu.sync_copy(src_hbm.at[pl.ds(base + c, CH)], src_v)
            pltpu.sync_copy(dst_hbm.at[pl.ds(base + c, CH)], dst_v)

            start_gath(0, rows_a, sem_a)
            start_gath(1, rows_b, sem_b)

            @pl.loop(0, CH - 2, step=2)
            def _(r):
                wait_gath(r, rows_a, sem_a)
                sc_a = pltpu.async_copy(rows_a, acc_sh.at[dst_v.at[r]],
                                        sem_sa, add=True)
                wait_gath(r + 1, rows_b, sem_b)
                sc_b = pltpu.async_copy(rows_b, acc_sh.at[dst_v.at[r + 1]],
                                        sem_sb, add=True)
                sc_a.wait()
                start_gath(r + 2, rows_a, sem_a)
                sc_b.wait()
                start_gath(r + 3, rows_b, sem_b)

            wait_gath(CH - 2, rows_a, sem_a)
            pltpu.sync_copy(rows_a, acc_sh.at[dst_v.at[CH - 2]], add=True)
            wait_gath(CH - 1, rows_b, sem_b)
            pltpu.sync_copy(rows_b, acc_sh.at[dst_v.at[CH - 1]], add=True)

        plsc.subcore_barrier()
        pltpu.sync_copy(
            acc_sh.at[pl.ds(rowbase, NPT)],
            out_hbm.at[cid].at[pl.ds(rowbase, NPT)],
        )

    return agg_kernel


BR = 640


def _dinv_block(degp_ref):
    i = pl.program_id(0)
    deg = degp_ref[0, i, :] + degp_ref[1, i, :] + 1.0
    return lax.rsqrt(deg)


def _mm1_body(x_ref, w_ref, o_ref):
    o_ref[...] = jnp.dot(x_ref[...], w_ref[...],
                         preferred_element_type=jnp.float32,
                         precision=lax.Precision.HIGHEST)


def _scale_body(h_ref, degp_ref, o_ref):
    dinv = _dinv_block(degp_ref)
    o_ref[...] = h_ref[...] * dinv[:, None]


def _layer_body(p_ref, g_ref, degp_ref, b_ref, w_ref, o_ref):
    dinv = _dinv_block(degp_ref)
    s = (p_ref[0] + p_ref[1] + g_ref[...]) * dinv[:, None] + b_ref[...]
    h = jnp.maximum(s, 0.0)
    o_ref[...] = jnp.dot(h, w_ref[...],
                         preferred_element_type=jnp.float32,
                         precision=lax.Precision.HIGHEST) * dinv[:, None]


def _layer_scale_body(p_ref, g_ref, degp_ref, b_ref, o_ref):
    dinv = _dinv_block(degp_ref)
    s = (p_ref[0] + p_ref[1] + g_ref[...]) * dinv[:, None] + b_ref[...]
    o_ref[...] = jnp.maximum(s, 0.0) * dinv[:, None]


def _final_body(p_ref, g_ref, degp_ref, w3_ref, b3_ref, x_ref, wr_ref,
                br_ref, o_ref):
    dinv = _dinv_block(degp_ref)
    t = (p_ref[0] + p_ref[1] + g_ref[...]) * dinv[:, None]
    conv = jnp.dot(t, w3_ref[...],
                   preferred_element_type=jnp.float32,
                   precision=lax.Precision.HIGHEST) + b3_ref[...]
    res = jnp.dot(x_ref[...], wr_ref[...],
                  preferred_element_type=jnp.float32,
                  precision=lax.Precision.HIGHEST) + br_ref[...]
    o_ref[...] = conv + res


def _row_spec(width):
    return pl.BlockSpec((BR, width), lambda i: (i, 0))


def _parts_spec(width):
    return pl.BlockSpec((NC, BR, width), lambda i: (0, i, 0))


def _full_spec(shape):
    nd = len(shape)
    return pl.BlockSpec(shape, lambda i: (0,) * nd)


def _tc_call(body, in_specs, out_width, grid=N_PAD // BR):
    return pl.pallas_call(
        body,
        grid=(grid,),
        in_specs=in_specs,
        out_specs=_row_spec(out_width),
        out_shape=jax.ShapeDtypeStruct((N_PAD, out_width), jnp.float32),
    )


def kernel(x, edge_index, W1, b1, W2, b2, W3, b3, Wr, br):
    d_in = x.shape[1]
    d_hid = W2.shape[0]
    d_out = W3.shape[1]

    src = edge_index[0].astype(jnp.int32)
    dst = edge_index[1].astype(jnp.int32)
    pad = N + (jnp.arange(E_PAD - E, dtype=jnp.int32) % (N_PAD - N))
    src2 = jnp.concatenate([src, pad]).reshape(ROWS, EROW)
    dst2 = jnp.concatenate([dst, pad]).reshape(ROWS, EROW)
    x_pad = jnp.zeros((N_PAD, d_in), jnp.float32).at[:N].set(x)

    degp = _make_deg_kernel()(dst2)

    h1 = _tc_call(_mm1_body, [_row_spec(d_in), _full_spec((d_in, d_hid))],
                  d_hid)(x_pad, W1)
    g1 = _tc_call(_scale_body, [_row_spec(d_hid), _degp_spec()],
                  d_hid)(h1, degp)
    parts1 = _make_agg_kernel(d_hid)(src2, dst2, g1)

    g2 = _tc_call(
        _layer_body,
        [_parts_spec(d_hid), _row_spec(d_hid), _degp_spec(),
         _full_spec((1, d_hid)), _full_spec((d_hid, d_hid))],
        d_hid)(parts1, g1, degp, b1.reshape(1, -1), W2)
    parts2 = _make_agg_kernel(d_hid)(src2, dst2, g2)

    g2b = _tc_call(
        _layer_scale_body,
        [_parts_spec(d_hid), _row_spec(d_hid), _degp_spec(),
         _full_spec((1, d_hid))],
        d_hid)(parts2, g2, degp, b2.reshape(1, -1))
    parts3 = _make_agg_kernel(d_hid)(src2, dst2, g2b)

    out = _tc_call(
        _final_body,
        [_parts_spec(d_hid), _row_spec(d_hid), _degp_spec(),
         _full_spec((d_hid, d_out)), _full_spec((1, d_out)), _row_spec(d_in),
         _full_spec((d_in, d_out)), _full_spec((1, d_out))],
        d_out)(parts3, g2b, degp, W3, b3.reshape(1, -1), x_pad, Wr,
               br.reshape(1, -1))
    return out[:N]


# speedup vs baseline: 22.5900x; 1.0035x over previous
import dataclasses
import functools

import jax
import jax.numpy as jnp
from jax import lax
from jax.experimental import pallas as pl
from jax.experimental.pallas import tpu as pltpu
from jax.experimental.pallas import tpu_sc as plsc

_SC_CP = pltpu.CompilerParams()
if "needs_layout_passes" in pltpu.CompilerParams.__dataclass_fields__:
    _SC_CP = dataclasses.replace(_SC_CP, needs_layout_passes=False)

NC = 2
NS = 16
LANES = 16
EROW = 128

N = 10000
E = 320000
N_PAD = 10240
ROWS = 2560
E_PAD = ROWS * EROW
RPT = ROWS // (NC * NS)
NPT = N_PAD // NS
def _degp_spec():
    return pl.BlockSpec((NC, NS, NPT), lambda i: (0, 0, 0))

_MESH = plsc.VectorSubcoreMesh(core_axis_name="c", subcore_axis_name="s")


def _zero_fill(ref, rows, width, value=0.0):
    vec = jnp.full((LANES,), value, jnp.float32)
    for i in range(rows):
        for j in range(width // LANES):
            ref[i, pl.ds(j * LANES, LANES)] = vec


def _make_deg_kernel():
    @functools.partial(
        pl.kernel,
        out_type=jax.ShapeDtypeStruct((NC, NS, NPT), jnp.float32),
        mesh=_MESH,
        scratch_types=[
            pltpu.VMEM((RPT, EROW), jnp.int32),
            pltpu.VMEM((N_PAD,), jnp.float32),
            pltpu.VMEM((NS, NPT), jnp.float32),
            pltpu.VMEM((NPT,), jnp.float32),
            pltpu.VMEM_SHARED((NS, N_PAD), jnp.float32),
        ],
        compiler_params=_SC_CP,
    )
    def deg_kernel(dst_hbm, out_hbm, dst_v, hist_v, red_v, sum_v, slab_sh):
        cid = lax.axis_index("c")
        sid = lax.axis_index("s")
        wid = cid * NS + sid

        pltpu.sync_copy(dst_hbm.at[pl.ds(wid * RPT, RPT)], dst_v)

        zvec = jnp.zeros((LANES,), jnp.float32)

        @pl.loop(0, N_PAD, step=LANES)
        def _(k):
            hist_v[pl.ds(k, LANES)] = zvec

        ones = jnp.ones((LANES,), jnp.float32)

        @pl.loop(0, RPT)
        def _(r):
            for g in range(EROW // LANES):
                iv = dst_v[r, pl.ds(g * LANES, LANES)]
                plsc.addupdate_scatter(hist_v, [iv], ones)

        pltpu.sync_copy(hist_v, slab_sh.at[sid])
        plsc.subcore_barrier()

        colbase = sid * NPT
        for rr in range(NS):
            pltpu.sync_copy(slab_sh.at[rr].at[pl.ds(colbase, NPT)],
                            red_v.at[rr])
        for c in range(NPT // LANES):
            sl = pl.ds(c * LANES, LANES)
            acc = red_v[0, sl]
            for rr in range(1, NS):
                acc = acc + red_v[rr, sl]
            sum_v[sl] = acc
        pltpu.sync_copy(sum_v, out_hbm.at[cid].at[sid])

    return deg_kernel


CH = 40


def _make_agg_kernel(feat):
    @functools.partial(
        pl.kernel,
        out_type=jax.ShapeDtypeStruct((NC, N_PAD, feat), jnp.float32),
        mesh=_MESH,
        scratch_types=[
            pltpu.VMEM((CH, EROW), jnp.int32),
            pltpu.VMEM((CH, EROW), jnp.int32),
            pltpu.VMEM((EROW, feat), jnp.float32),
            pltpu.VMEM((EROW, feat), jnp.float32),
            pltpu.VMEM_SHARED((N_PAD, feat), jnp.float32),
            pltpu.SemaphoreType.DMA,
            pltpu.SemaphoreType.DMA,
            pltpu.SemaphoreType.DMA,
            pltpu.SemaphoreType.DMA,
        ],
    )
    def agg_kernel(src_hbm, dst_hbm, g_hbm, out_hbm,
                   src_v, dst_v, rows_a, rows_b, acc_sh,
                   sem_a, sem_b, sem_sa, sem_sb):
        cid = lax.axis_index("c")
        sid = lax.axis_index("s")
        wid = cid * NS + sid
        base = wid * RPT

        _zero_fill(rows_a, EROW, feat, 0.0)
        rowbase = sid * NPT

        @pl.loop(0, NPT, step=EROW)
        def _(k):
            pltpu.sync_copy(rows_a, acc_sh.at[pl.ds(rowbase + k, EROW)])

        plsc.subcore_barrier()

        def start_gath(r, buf, sem):
            pltpu.make_async_copy(g_hbm.at[src_v.at[r]], buf, sem).start()

        def wait_gath(r, buf, sem):
            pltpu.make_async_copy(g_hbm.at[src_v.at[r]], buf, sem).wait()

        @pl.loop(0, RPT, step=CH)
        def _(c):
            pltpu.sync_copy(src_hbm.at[pl.ds(base + c, CH)], src_v)
            pltpu.sync_copy(dst_hbm.at[pl.ds(base + c, CH)], dst_v)

            start_gath(0, rows_a, sem_a)
            start_gath(1, rows_b, sem_b)

            @pl.loop(0, CH - 2, step=2)
            def _(r):
                wait_gath(r, rows_a, sem_a)
                sc_a = pltpu.async_copy(rows_a, acc_sh.at[dst_v.at[r]],
                                        sem_sa, add=True)
                wait_gath(r + 1, rows_b, sem_b)
                sc_b = pltpu.async_copy(rows_b, acc_sh.at[dst_v.at[r + 1]],
                                        sem_sb, add=True)
                sc_a.wait()
                start_gath(r + 2, rows_a, sem_a)
                sc_b.wait()
                start_gath(r + 3, rows_b, sem_b)

            wait_gath(CH - 2, rows_a, sem_a)
            pltpu.sync_copy(rows_a, acc_sh.at[dst_v.at[CH - 2]], add=True)
            wait_gath(CH - 1, rows_b, sem_b)
            pltpu.sync_copy(rows_b, acc_sh.at[dst_v.at[CH - 1]], add=True)

        plsc.subcore_barrier()
        pltpu.sync_copy(
            acc_sh.at[pl.ds(rowbase, NPT)],
            out_hbm.at[cid].at[pl.ds(rowbase, NPT)],
        )

    return agg_kernel


BR = 640


def _dinv_block(degp_ref):
    i = pl.program_id(0)
    deg = degp_ref[0, i, :] + degp_ref[1, i, :] + 1.0
    return lax.rsqrt(deg)


def _mm1_body(x_ref, w_ref, degp_ref, o_ref):
    dinv = _dinv_block(degp_ref)
    o_ref[...] = jnp.dot(x_ref[...], w_ref[...],
                         preferred_element_type=jnp.float32,
                         precision=lax.Precision.HIGHEST) * dinv[:, None]


def _layer_body(p_ref, g_ref, degp_ref, b_ref, w_ref, o_ref):
    dinv = _dinv_block(degp_ref)
    s = (p_ref[0] + p_ref[1] + g_ref[...]) * dinv[:, None] + b_ref[...]
    h = jnp.maximum(s, 0.0)
    o_ref[...] = jnp.dot(h, w_ref[...],
                         preferred_element_type=jnp.float32,
                         precision=lax.Precision.HIGHEST) * dinv[:, None]


def _layer_scale_body(p_ref, g_ref, degp_ref, b_ref, o_ref):
    dinv = _dinv_block(degp_ref)
    s = (p_ref[0] + p_ref[1] + g_ref[...]) * dinv[:, None] + b_ref[...]
    o_ref[...] = jnp.maximum(s, 0.0) * dinv[:, None]


def _final_body(p_ref, g_ref, degp_ref, w3_ref, b3_ref, x_ref, wr_ref,
                br_ref, o_ref):
    dinv = _dinv_block(degp_ref)
    t = (p_ref[0] + p_ref[1] + g_ref[...]) * dinv[:, None]
    conv = jnp.dot(t, w3_ref[...],
                   preferred_element_type=jnp.float32,
                   precision=lax.Precision.HIGHEST) + b3_ref[...]
    res = jnp.dot(x_ref[...], wr_ref[...],
                  preferred_element_type=jnp.float32,
                  precision=lax.Precision.HIGHEST) + br_ref[...]
    o_ref[...] = conv + res


def _row_spec(width):
    return pl.BlockSpec((BR, width), lambda i: (i, 0))


def _parts_spec(width):
    return pl.BlockSpec((NC, BR, width), lambda i: (0, i, 0))


def _full_spec(shape):
    nd = len(shape)
    return pl.BlockSpec(shape, lambda i: (0,) * nd)


def _tc_call(body, in_specs, out_width, grid=N_PAD // BR):
    return pl.pallas_call(
        body,
        grid=(grid,),
        in_specs=in_specs,
        out_specs=_row_spec(out_width),
        out_shape=jax.ShapeDtypeStruct((N_PAD, out_width), jnp.float32),
    )


def kernel(x, edge_index, W1, b1, W2, b2, W3, b3, Wr, br):
    d_in = x.shape[1]
    d_hid = W2.shape[0]
    d_out = W3.shape[1]

    src = edge_index[0].astype(jnp.int32)
    dst = edge_index[1].astype(jnp.int32)
    pad = N + (jnp.arange(E_PAD - E, dtype=jnp.int32) % (N_PAD - N))
    src2 = jnp.concatenate([src, pad]).reshape(ROWS, EROW)
    dst2 = jnp.concatenate([dst, pad]).reshape(ROWS, EROW)
    x_pad = jnp.zeros((N_PAD, d_in), jnp.float32).at[:N].set(x)

    degp = _make_deg_kernel()(dst2)

    g1 = _tc_call(_mm1_body,
                  [_row_spec(d_in), _full_spec((d_in, d_hid)), _degp_spec()],
                  d_hid)(x_pad, W1, degp)
    parts1 = _make_agg_kernel(d_hid)(src2, dst2, g1)

    g2 = _tc_call(
        _layer_body,
        [_parts_spec(d_hid), _row_spec(d_hid), _degp_spec(),
         _full_spec((1, d_hid)), _full_spec((d_hid, d_hid))],
        d_hid)(parts1, g1, degp, b1.reshape(1, -1), W2)
    parts2 = _make_agg_kernel(d_hid)(src2, dst2, g2)

    g2b = _tc_call(
        _layer_scale_body,
        [_parts_spec(d_hid), _row_spec(d_hid), _degp_spec(),
         _full_spec((1, d_hid))],
        d_hid)(parts2, g2, degp, b2.reshape(1, -1))
    parts3 = _make_agg_kernel(d_hid)(src2, dst2, g2b)

    out = _tc_call(
        _final_body,
        [_parts_spec(d_hid), _row_spec(d_hid), _degp_spec(),
         _full_spec((d_hid, d_out)), _full_spec((1, d_out)), _row_spec(d_in),
         _full_spec((d_in, d_out)), _full_spec((1, d_out))],
        d_out)(parts3, g2b, degp, W3, b3.reshape(1, -1), x_pad, Wr,
               br.reshape(1, -1))
    return out[:N]
